# baseline (speedup 1.0000x reference)
import functools

import jax
import jax.numpy as jnp
from jax import lax
from jax.experimental import pallas as pl
from jax.experimental.pallas import tpu as pltpu
from jax.experimental.pallas import tpu_sc as plsc

T = 2048
D_MODEL = 1024
MOE_FF = 512
SHARED_FF = 1024
N_EXPERTS = 8
B = 256
NT = 24
S = NT * B
NW = 32
ROWS_W = S // NW
TOK_W = T // NW


def _router_kernel(x_ref, gate_w_ref, meta_ref, cw_ref, meta2_ref):
    x = x_ref[...]
    logits = jnp.dot(x, gate_w_ref[...].T, preferred_element_type=jnp.float32)
    scores = jax.nn.softmax(logits, axis=-1)
    e_iota = lax.broadcasted_iota(jnp.int32, scores.shape, 1)
    w1 = jnp.max(scores, axis=-1, keepdims=True)
    a1 = jnp.argmax(scores, axis=-1)
    oh1 = (e_iota == a1[:, None])
    masked = jnp.where(oh1, -jnp.inf, scores)
    w2 = jnp.max(masked, axis=-1, keepdims=True)
    a2 = jnp.argmax(masked, axis=-1)
    oh2 = (e_iota == a2[:, None])
    denom = w1 + w2 + 1e-20
    c1 = (w1 / denom)[:, 0]
    c2 = (w2 / denom)[:, 0]

    oh = (oh1 | oh2).astype(jnp.bfloat16)
    ri = lax.broadcasted_iota(jnp.int32, (256, 256), 0)
    ci = lax.broadcasted_iota(jnp.int32, (256, 256), 1)
    ltri = (ri > ci).astype(jnp.bfloat16)
    carry = jnp.zeros((1, N_EXPERTS), dtype=jnp.float32)
    ranks = []
    for c in range(T // 256):
        oh_c = oh[c * 256:(c + 1) * 256]
        r_loc = jnp.dot(ltri, oh_c, preferred_element_type=jnp.float32)
        ranks.append(r_loc + carry)
        carry = carry + jnp.sum(oh_c.astype(jnp.float32), axis=0,
                                keepdims=True)
    ranks = jnp.concatenate(ranks, axis=0)
    counts = carry

    cnt_i = counts.astype(jnp.int32)
    padded = ((cnt_i + (B - 1)) >> 8) << 8
    tri8 = (lax.broadcasted_iota(jnp.int32, (8, 8), 0)
            <= lax.broadcasted_iota(jnp.int32, (8, 8), 1)).astype(jnp.float32)
    ends = jnp.dot(padded.astype(jnp.float32), tri8,
                   preferred_element_type=jnp.float32)
    offs = ends - padded.astype(jnp.float32)

    oh1f = oh1.astype(jnp.float32)
    oh2f = oh2.astype(jnp.float32)
    r1 = jnp.sum(ranks * oh1f, axis=1)
    r2 = jnp.sum(ranks * oh2f, axis=1)
    p1 = (r1 + jnp.sum(offs * oh1f, axis=1)).astype(jnp.int32)
    p2 = (r2 + jnp.sum(offs * oh2f, axis=1)).astype(jnp.int32)

    lane = lax.broadcasted_iota(jnp.int32, (T, 128), 1)
    meta_ref[...] = jnp.where(lane == 0, p1[:, None],
                              jnp.where(lane == 1, p2[:, None], 0))
    cw_ref[...] = jnp.where(lane == 0, c1[:, None],
                            jnp.where(lane == 1, c2[:, None], 0.0))

    lane8 = lax.broadcasted_iota(jnp.int32, (1, N_EXPERTS), 1)
    starts = lax.broadcasted_iota(jnp.int32, (1, 128), 1).astype(jnp.float32) * B
    te = jnp.zeros((1, 128), dtype=jnp.int32)
    for e in range(N_EXPERTS):
        end_e = jnp.sum(ends * (lane8 == e).astype(jnp.float32), axis=1,
                        keepdims=True)
        te = te + (starts >= end_e).astype(jnp.int32)
    te = jnp.minimum(te, N_EXPERTS - 1)
    nact = (jnp.sum(ends * (lane8 == N_EXPERTS - 1).astype(jnp.float32),
                    axis=1, keepdims=True) / B).astype(jnp.int32)
    row = lax.broadcasted_iota(jnp.int32, (8, 128), 0)
    meta2_ref[...] = jnp.where(row == 0, te, jnp.where(row == 1, nact, 0))


def _router(x, gate_w):
    return pl.pallas_call(
        _router_kernel,
        out_shape=(
            jax.ShapeDtypeStruct((T, 128), jnp.int32),
            jax.ShapeDtypeStruct((T, 128), jnp.float32),
            jax.ShapeDtypeStruct((8, 128), jnp.int32),
        ),
        compiler_params=pltpu.CompilerParams(
            vmem_limit_bytes=100 * 1024 * 1024,
        ),
    )(x, gate_w)


def _shared_kernel(x_ref, wsg_ref, wsu_ref, wsd_ref, out_ref):
    xb = x_ref[...].astype(jnp.bfloat16)
    gs = jnp.dot(xb, wsg_ref[...].T, preferred_element_type=jnp.float32)
    us = jnp.dot(xb, wsu_ref[...].T, preferred_element_type=jnp.float32)
    hs = (jax.nn.silu(gs) * us).astype(jnp.bfloat16)
    out_ref[...] = jnp.dot(hs, wsd_ref[...].T,
                           preferred_element_type=jnp.float32)


def _shared_mlp(x, wsg, wsu, wsd):
    tt = T // 4
    return pl.pallas_call(
        _shared_kernel,
        grid=(4,),
        in_specs=[
            pl.BlockSpec((tt, D_MODEL), lambda i: (i, 0)),
            pl.BlockSpec((SHARED_FF, D_MODEL), lambda i: (0, 0)),
            pl.BlockSpec((SHARED_FF, D_MODEL), lambda i: (0, 0)),
            pl.BlockSpec((D_MODEL, SHARED_FF), lambda i: (0, 0)),
        ],
        out_specs=pl.BlockSpec((tt, D_MODEL), lambda i: (i, 0)),
        out_shape=jax.ShapeDtypeStruct((T, D_MODEL), jnp.float32),
    )(x, wsg, wsu, wsd)


def _combine_kernel(cw_ref, og1_ref, og2_ref, sh_ref, y_ref):
    c1 = cw_ref[:, 0:1]
    c2 = cw_ref[:, 1:2]
    y_ref[...] = og1_ref[...] * c1 + og2_ref[...] * c2 + sh_ref[...]


def _combine(cw, og1, og2, shared):
    tt = T // 4
    return pl.pallas_call(
        _combine_kernel,
        grid=(4,),
        in_specs=[
            pl.BlockSpec((tt, 128), lambda i: (i, 0)),
            pl.BlockSpec((tt, D_MODEL), lambda i: (i, 0)),
            pl.BlockSpec((tt, D_MODEL), lambda i: (i, 0)),
            pl.BlockSpec((tt, D_MODEL), lambda i: (i, 0)),
        ],
        out_specs=pl.BlockSpec((tt, D_MODEL), lambda i: (i, 0)),
        out_shape=jax.ShapeDtypeStruct((T, D_MODEL), jnp.float32),
    )(cw, og1, og2, shared)


@functools.lru_cache(maxsize=None)
def _sc_mesh():
    return plsc.VectorSubcoreMesh(core_axis_name="c", subcore_axis_name="s")


def _sc_dispatch_body(p1_hbm, p2_hbm, fill_hbm, x_hbm, xs_hbm,
                      p1_v, p2_v, src_t, src_v, idx_v, rows_v, sem):
    cid = lax.axis_index("c")
    sid = lax.axis_index("s")
    wid = sid * 2 + cid

    @pl.when(sid == 0)
    def _():
        pltpu.sync_copy(p1_hbm, p1_v)
        pltpu.sync_copy(p2_hbm, p2_v)
        pltpu.sync_copy(fill_hbm, src_t)

        def body(j, carry):
            toks = lax.iota(jnp.int32, 16) + j * 16
            idx1 = p1_v[pl.ds(j * 16, 16)]
            plsc.store_scatter(src_t, [idx1], toks)
            idx2 = p2_v[pl.ds(j * 16, 16)]
            plsc.store_scatter(src_t, [idx2], toks)
            return carry

        lax.fori_loop(0, T // 16, body, 0)
        pltpu.sync_copy(src_t, src_v)

    plsc.subcore_barrier()

    for ch in range(ROWS_W // 96):
        base = wid * ROWS_W + ch * 96
        pltpu.sync_copy(src_v.at[pl.ds(base, 96)], idx_v)
        pltpu.async_copy(x_hbm.at[idx_v], rows_v, sem).wait()
        pltpu.sync_copy(rows_v, xs_hbm.at[pl.ds(base, 96)])


def _sc_dispatch(p1, p2, fill, x):
    k = functools.partial(
        pl.kernel, mesh=_sc_mesh(),
        out_type=jax.ShapeDtypeStruct((S, D_MODEL), jnp.float32),
        scratch_types=[
            pltpu.VMEM((T,), jnp.int32),
            pltpu.VMEM((T,), jnp.int32),
            pltpu.VMEM((S,), jnp.int32),
            pltpu.VMEM_SHARED((S,), jnp.int32),
            pltpu.VMEM((96,), jnp.int32),
            pltpu.VMEM((96, D_MODEL), jnp.float32),
            pltpu.SemaphoreType.DMA,
        ],
        compiler_params=pltpu.CompilerParams(needs_layout_passes=False),
    )(_sc_dispatch_body)
    return k(p1, p2, fill, x)


def _ffn_kernel(sarr_ref, xs_ref, wg_ref, wu_ref, wd_ref, o_ref):
    i = pl.program_id(0)
    nact = sarr_ref[32]

    @pl.when(i < nact)
    def _():
        xb = xs_ref[...].astype(jnp.bfloat16)
        g = jnp.dot(xb, wg_ref[0].T, preferred_element_type=jnp.float32)
        u = jnp.dot(xb, wu_ref[0].T, preferred_element_type=jnp.float32)
        h = (jax.nn.silu(g) * u).astype(jnp.bfloat16)
        o_ref[...] = jnp.dot(h, wd_ref[0].T,
                             preferred_element_type=jnp.float32)


def _ffn(sarr, xs, wg, wu, wd):
    spec = pltpu.PrefetchScalarGridSpec(
        num_scalar_prefetch=1,
        grid=(NT,),
        in_specs=[
            pl.BlockSpec((B, D_MODEL), lambda i, s: (i, 0)),
            pl.BlockSpec((1, MOE_FF, D_MODEL), lambda i, s: (s[i], 0, 0)),
            pl.BlockSpec((1, MOE_FF, D_MODEL), lambda i, s: (s[i], 0, 0)),
            pl.BlockSpec((1, D_MODEL, MOE_FF), lambda i, s: (s[i], 0, 0)),
        ],
        out_specs=pl.BlockSpec((B, D_MODEL), lambda i, s: (i, 0)),
    )
    return pl.pallas_call(
        _ffn_kernel,
        grid_spec=spec,
        out_shape=jax.ShapeDtypeStruct((S, D_MODEL), jnp.float32),
    )(sarr, xs, wg, wu, wd)


def _sc_gather_o_body(p1_hbm, p2_hbm, o_hbm, og1_hbm, og2_hbm, idx_v, rows_v,
                      sem):
    cid = lax.axis_index("c")
    sid = lax.axis_index("s")
    wid = sid * 2 + cid
    base = wid * TOK_W
    pltpu.sync_copy(p1_hbm.at[pl.ds(base, TOK_W)], idx_v)
    pltpu.async_copy(o_hbm.at[idx_v], rows_v, sem).wait()
    pltpu.sync_copy(rows_v, og1_hbm.at[pl.ds(base, TOK_W)])
    pltpu.sync_copy(p2_hbm.at[pl.ds(base, TOK_W)], idx_v)
    pltpu.async_copy(o_hbm.at[idx_v], rows_v, sem).wait()
    pltpu.sync_copy(rows_v, og2_hbm.at[pl.ds(base, TOK_W)])


def _sc_gather_o(p1, p2, o):
    k = functools.partial(
        pl.kernel, mesh=_sc_mesh(),
        out_type=(jax.ShapeDtypeStruct((T, D_MODEL), jnp.float32),
                  jax.ShapeDtypeStruct((T, D_MODEL), jnp.float32)),
        scratch_types=[
            pltpu.VMEM((TOK_W,), jnp.int32),
            pltpu.VMEM((TOK_W, D_MODEL), jnp.float32),
            pltpu.SemaphoreType.DMA,
        ],
    )(_sc_gather_o_body)
    return k(p1, p2, o)


@functools.partial(jax.jit, static_argnames=())
def kernel(hidden_states, gate_w, w_gate, w_up, w_down, ws_gate, ws_up, ws_down):
    orig_shape = hidden_states.shape
    x = hidden_states.reshape(-1, orig_shape[-1])

    wg = w_gate.astype(jnp.bfloat16)
    wu = w_up.astype(jnp.bfloat16)
    wd = w_down.astype(jnp.bfloat16)
    wsg = ws_gate.astype(jnp.bfloat16)
    wsu = ws_up.astype(jnp.bfloat16)
    wsd = ws_down.astype(jnp.bfloat16)

    meta, cw, meta2 = _router(x, gate_w)
    p1 = meta[:, 0]
    p2 = meta[:, 1]
    sarr = jnp.concatenate([meta2[0, :32], meta2[1, :32]])

    pad_fill = jnp.arange(S, dtype=jnp.int32) % T
    xs = _sc_dispatch(p1, p2, pad_fill, x)
    shared = _shared_mlp(x, wsg, wsu, wsd)
    o = _ffn(sarr, xs, wg, wu, wd)
    og1, og2 = _sc_gather_o(p1, p2, o)
    y = _combine(cw, og1, og2, shared)
    return y.reshape(orig_shape)


# speedup vs baseline: 1.0459x; 1.0459x over previous
import functools

import jax
import jax.numpy as jnp
from jax import lax
from jax.experimental import pallas as pl
from jax.experimental.pallas import tpu as pltpu
from jax.experimental.pallas import tpu_sc as plsc

T = 2048
D_MODEL = 1024
MOE_FF = 512
SHARED_FF = 1024
N_EXPERTS = 8
B = 256
NT = 24
S = NT * B
NW = 32
ROWS_W = S // NW
TOK_W = T // NW


def _router_kernel(x_ref, gate_w_ref, meta_ref, cw_ref, meta2_ref):
    x = x_ref[...]
    logits = jnp.dot(x, gate_w_ref[...].T, preferred_element_type=jnp.float32)
    scores = jax.nn.softmax(logits, axis=-1)
    e_iota = lax.broadcasted_iota(jnp.int32, scores.shape, 1)
    w1 = jnp.max(scores, axis=-1, keepdims=True)
    a1 = jnp.argmax(scores, axis=-1)
    oh1 = (e_iota == a1[:, None])
    masked = jnp.where(oh1, -jnp.inf, scores)
    w2 = jnp.max(masked, axis=-1, keepdims=True)
    a2 = jnp.argmax(masked, axis=-1)
    oh2 = (e_iota == a2[:, None])
    denom = w1 + w2 + 1e-20
    c1 = (w1 / denom)[:, 0]
    c2 = (w2 / denom)[:, 0]

    oh = (oh1 | oh2).astype(jnp.bfloat16)
    ri = lax.broadcasted_iota(jnp.int32, (256, 256), 0)
    ci = lax.broadcasted_iota(jnp.int32, (256, 256), 1)
    ltri = (ri > ci).astype(jnp.bfloat16)
    carry = jnp.zeros((1, N_EXPERTS), dtype=jnp.float32)
    ranks = []
    for c in range(T // 256):
        oh_c = oh[c * 256:(c + 1) * 256]
        r_loc = jnp.dot(ltri, oh_c, preferred_element_type=jnp.float32)
        ranks.append(r_loc + carry)
        carry = carry + jnp.sum(oh_c.astype(jnp.float32), axis=0,
                                keepdims=True)
    ranks = jnp.concatenate(ranks, axis=0)
    counts = carry

    cnt_i = counts.astype(jnp.int32)
    padded = ((cnt_i + (B - 1)) >> 8) << 8
    tri8 = (lax.broadcasted_iota(jnp.int32, (8, 8), 0)
            <= lax.broadcasted_iota(jnp.int32, (8, 8), 1)).astype(jnp.float32)
    ends = jnp.dot(padded.astype(jnp.float32), tri8,
                   preferred_element_type=jnp.float32)
    offs = ends - padded.astype(jnp.float32)

    oh1f = oh1.astype(jnp.float32)
    oh2f = oh2.astype(jnp.float32)
    r1 = jnp.sum(ranks * oh1f, axis=1)
    r2 = jnp.sum(ranks * oh2f, axis=1)
    p1 = (r1 + jnp.sum(offs * oh1f, axis=1)).astype(jnp.int32)
    p2 = (r2 + jnp.sum(offs * oh2f, axis=1)).astype(jnp.int32)

    lane = lax.broadcasted_iota(jnp.int32, (T, 128), 1)
    meta_ref[...] = jnp.where(lane == 0, p1[:, None],
                              jnp.where(lane == 1, p2[:, None], 0))
    cw_ref[...] = jnp.where(lane == 0, c1[:, None],
                            jnp.where(lane == 1, c2[:, None], 0.0))

    lane8 = lax.broadcasted_iota(jnp.int32, (1, N_EXPERTS), 1)
    starts = lax.broadcasted_iota(jnp.int32, (1, 128), 1).astype(jnp.float32) * B
    te = jnp.zeros((1, 128), dtype=jnp.int32)
    for e in range(N_EXPERTS):
        end_e = jnp.sum(ends * (lane8 == e).astype(jnp.float32), axis=1,
                        keepdims=True)
        te = te + (starts >= end_e).astype(jnp.int32)
    te = jnp.minimum(te, N_EXPERTS - 1)
    nact = (jnp.sum(ends * (lane8 == N_EXPERTS - 1).astype(jnp.float32),
                    axis=1, keepdims=True) / B).astype(jnp.int32)
    row = lax.broadcasted_iota(jnp.int32, (8, 128), 0)
    meta2_ref[...] = jnp.where(row == 0, te, jnp.where(row == 1, nact, 0))


def _router(x, gate_w):
    return pl.pallas_call(
        _router_kernel,
        out_shape=(
            jax.ShapeDtypeStruct((T, 128), jnp.int32),
            jax.ShapeDtypeStruct((T, 128), jnp.float32),
            jax.ShapeDtypeStruct((8, 128), jnp.int32),
        ),
        compiler_params=pltpu.CompilerParams(
            vmem_limit_bytes=100 * 1024 * 1024,
        ),
    )(x, gate_w)


def _shared_combine_kernel(x_ref, wsg_ref, wsu_ref, wsd_ref,
                           cw_ref, og1_ref, og2_ref, y_ref):
    xb = x_ref[...].astype(jnp.bfloat16)
    gs = jnp.dot(xb, wsg_ref[...].T, preferred_element_type=jnp.float32)
    us = jnp.dot(xb, wsu_ref[...].T, preferred_element_type=jnp.float32)
    hs = (jax.nn.silu(gs) * us).astype(jnp.bfloat16)
    shared = jnp.dot(hs, wsd_ref[...].T, preferred_element_type=jnp.float32)
    c1 = cw_ref[:, 0:1]
    c2 = cw_ref[:, 1:2]
    y_ref[...] = og1_ref[...] * c1 + og2_ref[...] * c2 + shared


def _shared_combine(x, wsg, wsu, wsd, cw, og1, og2):
    tt = T // 4
    return pl.pallas_call(
        _shared_combine_kernel,
        grid=(4,),
        in_specs=[
            pl.BlockSpec((tt, D_MODEL), lambda i: (i, 0)),
            pl.BlockSpec((SHARED_FF, D_MODEL), lambda i: (0, 0)),
            pl.BlockSpec((SHARED_FF, D_MODEL), lambda i: (0, 0)),
            pl.BlockSpec((D_MODEL, SHARED_FF), lambda i: (0, 0)),
            pl.BlockSpec((tt, 128), lambda i: (i, 0)),
            pl.BlockSpec((tt, D_MODEL), lambda i: (i, 0)),
            pl.BlockSpec((tt, D_MODEL), lambda i: (i, 0)),
        ],
        out_specs=pl.BlockSpec((tt, D_MODEL), lambda i: (i, 0)),
        out_shape=jax.ShapeDtypeStruct((T, D_MODEL), jnp.float32),
    )(x, wsg, wsu, wsd, cw, og1, og2)


@functools.lru_cache(maxsize=None)
def _sc_mesh():
    return plsc.VectorSubcoreMesh(core_axis_name="c", subcore_axis_name="s")


def _sc_dispatch_body(p1_hbm, p2_hbm, fill_hbm, x_hbm, xs_hbm,
                      p1_v, p2_v, src_t, src_v, idx_v, rows_v, sem):
    cid = lax.axis_index("c")
    sid = lax.axis_index("s")
    wid = sid * 2 + cid

    @pl.when(sid == 0)
    def _():
        pltpu.sync_copy(p1_hbm, p1_v)
        pltpu.sync_copy(p2_hbm, p2_v)
        pltpu.sync_copy(fill_hbm, src_t)

        def body(j, carry):
            toks = lax.iota(jnp.int32, 16) + j * 16
            idx1 = p1_v[pl.ds(j * 16, 16)]
            plsc.store_scatter(src_t, [idx1], toks)
            idx2 = p2_v[pl.ds(j * 16, 16)]
            plsc.store_scatter(src_t, [idx2], toks)
            return carry

        lax.fori_loop(0, T // 16, body, 0)
        pltpu.sync_copy(src_t, src_v)

    plsc.subcore_barrier()

    for ch in range(ROWS_W // 96):
        base = wid * ROWS_W + ch * 96
        pltpu.sync_copy(src_v.at[pl.ds(base, 96)], idx_v)
        pltpu.async_copy(x_hbm.at[idx_v], rows_v, sem).wait()
        pltpu.sync_copy(rows_v, xs_hbm.at[pl.ds(base, 96)])


def _sc_dispatch(p1, p2, fill, x):
    k = functools.partial(
        pl.kernel, mesh=_sc_mesh(),
        out_type=jax.ShapeDtypeStruct((S, D_MODEL), jnp.float32),
        scratch_types=[
            pltpu.VMEM((T,), jnp.int32),
            pltpu.VMEM((T,), jnp.int32),
            pltpu.VMEM((S,), jnp.int32),
            pltpu.VMEM_SHARED((S,), jnp.int32),
            pltpu.VMEM((96,), jnp.int32),
            pltpu.VMEM((96, D_MODEL), jnp.float32),
            pltpu.SemaphoreType.DMA,
        ],
        compiler_params=pltpu.CompilerParams(needs_layout_passes=False),
    )(_sc_dispatch_body)
    return k(p1, p2, fill, x)


def _ffn_kernel(sarr_ref, xs_ref, wg_ref, wu_ref, wd_ref, o_ref):
    i = pl.program_id(0)
    nact = sarr_ref[32]

    @pl.when(i < nact)
    def _():
        xb = xs_ref[...].astype(jnp.bfloat16)
        g = jnp.dot(xb, wg_ref[0].T, preferred_element_type=jnp.float32)
        u = jnp.dot(xb, wu_ref[0].T, preferred_element_type=jnp.float32)
        h = (jax.nn.silu(g) * u).astype(jnp.bfloat16)
        o_ref[...] = jnp.dot(h, wd_ref[0].T,
                             preferred_element_type=jnp.float32)


def _ffn(sarr, xs, wg, wu, wd):
    spec = pltpu.PrefetchScalarGridSpec(
        num_scalar_prefetch=1,
        grid=(NT,),
        in_specs=[
            pl.BlockSpec((B, D_MODEL), lambda i, s: (i, 0)),
            pl.BlockSpec((1, MOE_FF, D_MODEL), lambda i, s: (s[i], 0, 0)),
            pl.BlockSpec((1, MOE_FF, D_MODEL), lambda i, s: (s[i], 0, 0)),
            pl.BlockSpec((1, D_MODEL, MOE_FF), lambda i, s: (s[i], 0, 0)),
        ],
        out_specs=pl.BlockSpec((B, D_MODEL), lambda i, s: (i, 0)),
    )
    return pl.pallas_call(
        _ffn_kernel,
        grid_spec=spec,
        out_shape=jax.ShapeDtypeStruct((S, D_MODEL), jnp.float32),
    )(sarr, xs, wg, wu, wd)


def _sc_gather_o_body(p1_hbm, p2_hbm, o_hbm, og1_hbm, og2_hbm, idx_v, rows_v,
                      sem):
    cid = lax.axis_index("c")
    sid = lax.axis_index("s")
    wid = sid * 2 + cid
    base = wid * TOK_W
    pltpu.sync_copy(p1_hbm.at[pl.ds(base, TOK_W)], idx_v)
    pltpu.async_copy(o_hbm.at[idx_v], rows_v, sem).wait()
    pltpu.sync_copy(rows_v, og1_hbm.at[pl.ds(base, TOK_W)])
    pltpu.sync_copy(p2_hbm.at[pl.ds(base, TOK_W)], idx_v)
    pltpu.async_copy(o_hbm.at[idx_v], rows_v, sem).wait()
    pltpu.sync_copy(rows_v, og2_hbm.at[pl.ds(base, TOK_W)])


def _sc_gather_o(p1, p2, o):
    k = functools.partial(
        pl.kernel, mesh=_sc_mesh(),
        out_type=(jax.ShapeDtypeStruct((T, D_MODEL), jnp.float32),
                  jax.ShapeDtypeStruct((T, D_MODEL), jnp.float32)),
        scratch_types=[
            pltpu.VMEM((TOK_W,), jnp.int32),
            pltpu.VMEM((TOK_W, D_MODEL), jnp.float32),
            pltpu.SemaphoreType.DMA,
        ],
    )(_sc_gather_o_body)
    return k(p1, p2, o)


@functools.partial(jax.jit, static_argnames=())
def kernel(hidden_states, gate_w, w_gate, w_up, w_down, ws_gate, ws_up, ws_down):
    orig_shape = hidden_states.shape
    x = hidden_states.reshape(-1, orig_shape[-1])

    wg = w_gate.astype(jnp.bfloat16)
    wu = w_up.astype(jnp.bfloat16)
    wd = w_down.astype(jnp.bfloat16)
    wsg = ws_gate.astype(jnp.bfloat16)
    wsu = ws_up.astype(jnp.bfloat16)
    wsd = ws_down.astype(jnp.bfloat16)

    meta, cw, meta2 = _router(x, gate_w)
    p1 = meta[:, 0]
    p2 = meta[:, 1]
    sarr = jnp.concatenate([meta2[0, :32], meta2[1, :32]])

    pad_fill = jnp.arange(S, dtype=jnp.int32) % T
    xs = _sc_dispatch(p1, p2, pad_fill, x)
    o = _ffn(sarr, xs, wg, wu, wd)
    og1, og2 = _sc_gather_o(p1, p2, o)
    y = _shared_combine(x, wsg, wsu, wsd, cw, og1, og2)
    return y.reshape(orig_shape)


# speedup vs baseline: 1.0535x; 1.0073x over previous
import functools

import jax
import jax.numpy as jnp
from jax import lax
from jax.experimental import pallas as pl
from jax.experimental.pallas import tpu as pltpu
from jax.experimental.pallas import tpu_sc as plsc

T = 2048
D_MODEL = 1024
MOE_FF = 512
SHARED_FF = 1024
N_EXPERTS = 8
B = 256
NT = 24
S = NT * B
NW = 32
ROWS_W = S // NW
TOK_W = T // NW


def _router_kernel(x_ref, gate_w_ref, meta_ref, cw_ref, meta2_ref):
    x = x_ref[...]
    logits = jnp.dot(x, gate_w_ref[...].T, preferred_element_type=jnp.float32)
    scores = jax.nn.softmax(logits, axis=-1)
    e_iota = lax.broadcasted_iota(jnp.int32, scores.shape, 1)
    w1 = jnp.max(scores, axis=-1, keepdims=True)
    a1 = jnp.argmax(scores, axis=-1)
    oh1 = (e_iota == a1[:, None])
    masked = jnp.where(oh1, -jnp.inf, scores)
    w2 = jnp.max(masked, axis=-1, keepdims=True)
    a2 = jnp.argmax(masked, axis=-1)
    oh2 = (e_iota == a2[:, None])
    denom = w1 + w2 + 1e-20
    c1 = (w1 / denom)[:, 0]
    c2 = (w2 / denom)[:, 0]

    oh = (oh1 | oh2).astype(jnp.bfloat16)
    ri = lax.broadcasted_iota(jnp.int32, (256, 256), 0)
    ci = lax.broadcasted_iota(jnp.int32, (256, 256), 1)
    ltri = (ri > ci).astype(jnp.bfloat16)
    carry = jnp.zeros((1, N_EXPERTS), dtype=jnp.float32)
    ranks = []
    for c in range(T // 256):
        oh_c = oh[c * 256:(c + 1) * 256]
        r_loc = jnp.dot(ltri, oh_c, preferred_element_type=jnp.float32)
        ranks.append(r_loc + carry)
        carry = carry + jnp.sum(oh_c.astype(jnp.float32), axis=0,
                                keepdims=True)
    ranks = jnp.concatenate(ranks, axis=0)
    counts = carry

    cnt_i = counts.astype(jnp.int32)
    padded = ((cnt_i + (B - 1)) >> 8) << 8
    tri8 = (lax.broadcasted_iota(jnp.int32, (8, 8), 0)
            <= lax.broadcasted_iota(jnp.int32, (8, 8), 1)).astype(jnp.float32)
    ends = jnp.dot(padded.astype(jnp.float32), tri8,
                   preferred_element_type=jnp.float32)
    offs = ends - padded.astype(jnp.float32)

    oh1f = oh1.astype(jnp.float32)
    oh2f = oh2.astype(jnp.float32)
    r1 = jnp.sum(ranks * oh1f, axis=1)
    r2 = jnp.sum(ranks * oh2f, axis=1)
    p1 = (r1 + jnp.sum(offs * oh1f, axis=1)).astype(jnp.int32)
    p2 = (r2 + jnp.sum(offs * oh2f, axis=1)).astype(jnp.int32)

    lane = lax.broadcasted_iota(jnp.int32, (T, 128), 1)
    meta_ref[...] = jnp.where(lane == 0, p1[:, None],
                              jnp.where(lane == 1, p2[:, None], 0))
    cw_ref[...] = jnp.where(lane == 0, c1[:, None],
                            jnp.where(lane == 1, c2[:, None], 0.0))

    lane8 = lax.broadcasted_iota(jnp.int32, (1, N_EXPERTS), 1)
    starts = lax.broadcasted_iota(jnp.int32, (1, 128), 1).astype(jnp.float32) * B
    te = jnp.zeros((1, 128), dtype=jnp.int32)
    for e in range(N_EXPERTS):
        end_e = jnp.sum(ends * (lane8 == e).astype(jnp.float32), axis=1,
                        keepdims=True)
        te = te + (starts >= end_e).astype(jnp.int32)
    te = jnp.minimum(te, N_EXPERTS - 1)
    nact = (jnp.sum(ends * (lane8 == N_EXPERTS - 1).astype(jnp.float32),
                    axis=1, keepdims=True) / B).astype(jnp.int32)
    row = lax.broadcasted_iota(jnp.int32, (8, 128), 0)
    meta2_ref[...] = jnp.where(row == 0, te, jnp.where(row == 1, nact, 0))


def _router(x, gate_w):
    return pl.pallas_call(
        _router_kernel,
        out_shape=(
            jax.ShapeDtypeStruct((T, 128), jnp.int32),
            jax.ShapeDtypeStruct((T, 128), jnp.float32),
            jax.ShapeDtypeStruct((8, 128), jnp.int32),
        ),
        compiler_params=pltpu.CompilerParams(
            vmem_limit_bytes=100 * 1024 * 1024,
        ),
    )(x, gate_w)


def _shared_combine_kernel(x_ref, wsg_ref, wsu_ref, wsd_ref,
                           cw_ref, og1_ref, og2_ref, y_ref):
    xb = x_ref[...].astype(jnp.bfloat16)
    gs = jnp.dot(xb, wsg_ref[...].T, preferred_element_type=jnp.float32)
    us = jnp.dot(xb, wsu_ref[...].T, preferred_element_type=jnp.float32)
    hs = (jax.nn.silu(gs) * us).astype(jnp.bfloat16)
    shared = jnp.dot(hs, wsd_ref[...].T, preferred_element_type=jnp.float32)
    c1 = cw_ref[:, 0:1]
    c2 = cw_ref[:, 1:2]
    y_ref[...] = og1_ref[...] * c1 + og2_ref[...] * c2 + shared


def _shared_combine(x, wsg, wsu, wsd, cw, og1, og2):
    tt = T // 4
    return pl.pallas_call(
        _shared_combine_kernel,
        grid=(4,),
        in_specs=[
            pl.BlockSpec((tt, D_MODEL), lambda i: (i, 0)),
            pl.BlockSpec((SHARED_FF, D_MODEL), lambda i: (0, 0)),
            pl.BlockSpec((SHARED_FF, D_MODEL), lambda i: (0, 0)),
            pl.BlockSpec((D_MODEL, SHARED_FF), lambda i: (0, 0)),
            pl.BlockSpec((tt, 128), lambda i: (i, 0)),
            pl.BlockSpec((tt, D_MODEL), lambda i: (i, 0)),
            pl.BlockSpec((tt, D_MODEL), lambda i: (i, 0)),
        ],
        out_specs=pl.BlockSpec((tt, D_MODEL), lambda i: (i, 0)),
        out_shape=jax.ShapeDtypeStruct((T, D_MODEL), jnp.float32),
    )(x, wsg, wsu, wsd, cw, og1, og2)


@functools.lru_cache(maxsize=None)
def _sc_mesh():
    return plsc.VectorSubcoreMesh(core_axis_name="c", subcore_axis_name="s")


def _sc_dispatch_body(p1_hbm, p2_hbm, fill_hbm, sarr_hbm, x_hbm, xs_hbm,
                      p1_v, p2_v, src_t, src_v, idx_v, rows_v, nact_v, sem):
    cid = lax.axis_index("c")
    sid = lax.axis_index("s")
    wid = sid * 2 + cid
    pltpu.sync_copy(sarr_hbm.at[pl.ds(32, 16)], nact_v)
    n_slots = lax.reduce_max(nact_v[...], axes=(0,)) * B

    @pl.when(sid == 0)
    def _():
        pltpu.sync_copy(p1_hbm, p1_v)
        pltpu.sync_copy(p2_hbm, p2_v)
        pltpu.sync_copy(fill_hbm, src_t)

        def body(j, carry):
            toks = lax.iota(jnp.int32, 16) + j * 16
            idx1 = p1_v[pl.ds(j * 16, 16)]
            plsc.store_scatter(src_t, [idx1], toks)
            idx2 = p2_v[pl.ds(j * 16, 16)]
            plsc.store_scatter(src_t, [idx2], toks)
            return carry

        lax.fori_loop(0, T // 16, body, 0)
        pltpu.sync_copy(src_t, src_v)

    plsc.subcore_barrier()

    for ch in range(ROWS_W // 96):
        base = wid * ROWS_W + ch * 96

        @pl.when(base < n_slots)
        def _():
            pltpu.sync_copy(src_v.at[pl.ds(base, 96)], idx_v)
            pltpu.async_copy(x_hbm.at[idx_v], rows_v, sem).wait()
            pltpu.sync_copy(rows_v, xs_hbm.at[pl.ds(base, 96)])


def _sc_dispatch(p1, p2, fill, sarr, x):
    k = functools.partial(
        pl.kernel, mesh=_sc_mesh(),
        out_type=jax.ShapeDtypeStruct((S, D_MODEL), jnp.float32),
        scratch_types=[
            pltpu.VMEM((T,), jnp.int32),
            pltpu.VMEM((T,), jnp.int32),
            pltpu.VMEM((S,), jnp.int32),
            pltpu.VMEM_SHARED((S,), jnp.int32),
            pltpu.VMEM((96,), jnp.int32),
            pltpu.VMEM((96, D_MODEL), jnp.float32),
            pltpu.VMEM((16,), jnp.int32),
            pltpu.SemaphoreType.DMA,
        ],
        compiler_params=pltpu.CompilerParams(needs_layout_passes=False),
    )(_sc_dispatch_body)
    return k(p1, p2, fill, sarr, x)


def _ffn_kernel(sarr_ref, xs_ref, wg_ref, wu_ref, wd_ref, o_ref):
    i = pl.program_id(0)
    nact = sarr_ref[32]

    @pl.when(i < nact)
    def _():
        xb = xs_ref[...].astype(jnp.bfloat16)
        g = jnp.dot(xb, wg_ref[0].T, preferred_element_type=jnp.float32)
        u = jnp.dot(xb, wu_ref[0].T, preferred_element_type=jnp.float32)
        h = (jax.nn.silu(g) * u).astype(jnp.bfloat16)
        o_ref[...] = jnp.dot(h, wd_ref[0].T,
                             preferred_element_type=jnp.float32)


def _ffn(sarr, xs, wg, wu, wd):
    spec = pltpu.PrefetchScalarGridSpec(
        num_scalar_prefetch=1,
        grid=(NT,),
        in_specs=[
            pl.BlockSpec((B, D_MODEL), lambda i, s: (i, 0)),
            pl.BlockSpec((1, MOE_FF, D_MODEL), lambda i, s: (s[i], 0, 0)),
            pl.BlockSpec((1, MOE_FF, D_MODEL), lambda i, s: (s[i], 0, 0)),
            pl.BlockSpec((1, D_MODEL, MOE_FF), lambda i, s: (s[i], 0, 0)),
        ],
        out_specs=pl.BlockSpec((B, D_MODEL), lambda i, s: (i, 0)),
    )
    return pl.pallas_call(
        _ffn_kernel,
        grid_spec=spec,
        out_shape=jax.ShapeDtypeStruct((S, D_MODEL), jnp.float32),
    )(sarr, xs, wg, wu, wd)


def _sc_gather_o_body(p1_hbm, p2_hbm, o_hbm, og1_hbm, og2_hbm, idx_v, rows_v,
                      sem):
    cid = lax.axis_index("c")
    sid = lax.axis_index("s")
    wid = sid * 2 + cid
    base = wid * TOK_W
    pltpu.sync_copy(p1_hbm.at[pl.ds(base, TOK_W)], idx_v)
    pltpu.async_copy(o_hbm.at[idx_v], rows_v, sem).wait()
    pltpu.sync_copy(rows_v, og1_hbm.at[pl.ds(base, TOK_W)])
    pltpu.sync_copy(p2_hbm.at[pl.ds(base, TOK_W)], idx_v)
    pltpu.async_copy(o_hbm.at[idx_v], rows_v, sem).wait()
    pltpu.sync_copy(rows_v, og2_hbm.at[pl.ds(base, TOK_W)])


def _sc_gather_o(p1, p2, o):
    k = functools.partial(
        pl.kernel, mesh=_sc_mesh(),
        out_type=(jax.ShapeDtypeStruct((T, D_MODEL), jnp.float32),
                  jax.ShapeDtypeStruct((T, D_MODEL), jnp.float32)),
        scratch_types=[
            pltpu.VMEM((TOK_W,), jnp.int32),
            pltpu.VMEM((TOK_W, D_MODEL), jnp.float32),
            pltpu.SemaphoreType.DMA,
        ],
    )(_sc_gather_o_body)
    return k(p1, p2, o)


@functools.partial(jax.jit, static_argnames=())
def kernel(hidden_states, gate_w, w_gate, w_up, w_down, ws_gate, ws_up, ws_down):
    orig_shape = hidden_states.shape
    x = hidden_states.reshape(-1, orig_shape[-1])

    wg = w_gate.astype(jnp.bfloat16)
    wu = w_up.astype(jnp.bfloat16)
    wd = w_down.astype(jnp.bfloat16)
    wsg = ws_gate.astype(jnp.bfloat16)
    wsu = ws_up.astype(jnp.bfloat16)
    wsd = ws_down.astype(jnp.bfloat16)

    meta, cw, meta2 = _router(x, gate_w)
    p1 = meta[:, 0]
    p2 = meta[:, 1]
    sarr = jnp.concatenate([meta2[0, :32], meta2[1, :32]])

    pad_fill = jnp.arange(S, dtype=jnp.int32) % T
    xs = _sc_dispatch(p1, p2, pad_fill, sarr, x)
    o = _ffn(sarr, xs, wg, wu, wd)
    og1, og2 = _sc_gather_o(p1, p2, o)
    y = _shared_combine(x, wsg, wsu, wsd, cw, og1, og2)
    return y.reshape(orig_shape)
